# Initial kernel scaffold; baseline (speedup 1.0000x reference)
import jax
import jax.numpy as jnp
from jax.experimental import pallas as pl


def kernel(pseduo_x, rbf, rsh, edge_index, Wq, Wk, Wp, Wrbf, w_cg, Wv, Wmsg, ln_g, ln_b, W1, b1, W2, b2):
    raise NotImplementedError("write your pallas kernel here")


# speedup vs baseline: 9.8609x; 9.8609x over previous
import jax
import jax.numpy as jnp
from jax.experimental import pallas as pl

_N = 10000
_E = 320000
_D = 128
_H = 8
_M = 16
_ED = 16
_NB = 16
_EB = 4000
_BN = 1000


def _edge_body(xs_ref, xd_ref, rbf_ref, rsh_ref, wq_ref, wk_ref, wp_ref,
               wrbf_ref, wcg_ref, wv_ref, out_ref):
    xs = xs_ref[...]
    xd = xd_ref[...]
    cat = jnp.concatenate([xs, xd], axis=1)
    q = jnp.dot(xs, wq_ref[...])
    k = jnp.dot(cat, wk_ref[...])
    edge = rsh_ref[...] * jnp.dot(rbf_ref[...], wrbf_ref[...])
    sph = edge * jnp.dot(cat, wp_ref[...])
    mixed = wcg_ref[...] * sph * sph
    v = jnp.dot(mixed, wv_ref[...])
    prod = q * k
    heads = [jnp.sum(prod[:, h * _M:(h + 1) * _M], axis=1, keepdims=True)
             for h in range(_H)]
    logits = jnp.concatenate(heads, axis=1) * 0.25
    ex = jnp.exp(jnp.clip(logits, -70.0, 70.0))
    ex_rep = jnp.concatenate(
        [jnp.broadcast_to(ex[:, h:h + 1], (ex.shape[0], _M))
         for h in range(_H)], axis=1)
    out_ref[:, 0:_D] = ex_rep * v
    out_ref[:, _D:_D + _H] = ex
    out_ref[:, _D + _H:] = jnp.zeros_like(ex)


def _edge_pipeline(xs, xd, rbf, rsh, Wq, Wk, Wp, Wrbf, w_cg, Wv):
    e = xs.shape[0]
    wide = lambda: pl.BlockSpec((_EB, _D), lambda i: (i, 0))
    thin = lambda: pl.BlockSpec((_EB, _ED), lambda i: (i, 0))
    return pl.pallas_call(
        _edge_body,
        grid=(e // _EB,),
        in_specs=[wide(), wide(), thin(), thin(),
                  pl.BlockSpec((_D, _D), lambda i: (0, 0)),
                  pl.BlockSpec((2 * _D, _D), lambda i: (0, 0)),
                  pl.BlockSpec((2 * _D, _ED), lambda i: (0, 0)),
                  pl.BlockSpec((_NB, _ED), lambda i: (0, 0)),
                  pl.BlockSpec((1, _ED), lambda i: (0, 0)),
                  pl.BlockSpec((_ED, _D), lambda i: (0, 0))],
        out_specs=pl.BlockSpec((_EB, 144), lambda i: (i, 0)),
        out_shape=jax.ShapeDtypeStruct((e, 144), jnp.float32),
    )(xs, xd, rbf, rsh, Wq, Wk, Wp, Wrbf, w_cg.reshape(1, _ED), Wv)


def _final_body(acc_ref, wmsg_ref, lng_ref, lnb_ref, w1_ref, b1_ref,
                w2_ref, b2_ref, out_ref):
    acc = acc_ref[...]
    msum = acc[:, 0:_D]
    den = acc[:, _D:_D + _H]
    den_rep = jnp.concatenate(
        [jnp.broadcast_to(den[:, h:h + 1], (den.shape[0], _M))
         for h in range(_H)], axis=1)
    msg = msum / (den_rep + 1e-16)
    msg = jnp.dot(msg, wmsg_ref[...])
    x0 = jnp.abs(msg)
    mu = jnp.mean(x0, axis=-1, keepdims=True)
    var = jnp.mean((x0 - mu) ** 2, axis=-1, keepdims=True)
    x1 = (x0 - mu) / jnp.sqrt(var + 1e-5) * lng_ref[...] + lnb_ref[...]
    x2 = msg / (x0 + 1e-6)
    s = jnp.dot(x1, w1_ref[...]) + b1_ref[...]
    s = s * jax.nn.sigmoid(s)
    s = jnp.dot(s, w2_ref[...]) + b2_ref[...]
    s = s * jax.nn.sigmoid(s)
    out_ref[...] = x2 * s


def _node_finalize(acc, Wmsg, ln_g, ln_b, W1, b1, W2, b2):
    n = acc.shape[0]
    rep = pl.BlockSpec((_D, _D), lambda i: (0, 0))
    vec = pl.BlockSpec((1, _D), lambda i: (0, 0))
    return pl.pallas_call(
        _final_body,
        grid=(n // _BN,),
        in_specs=[pl.BlockSpec((_BN, 144), lambda i: (i, 0)),
                  rep, vec, vec, rep, vec, rep, vec],
        out_specs=pl.BlockSpec((_BN, _D), lambda i: (i, 0)),
        out_shape=jax.ShapeDtypeStruct((n, _D), jnp.float32),
    )(acc, Wmsg, ln_g.reshape(1, _D), ln_b.reshape(1, _D),
      W1, b1.reshape(1, _D), W2, b2.reshape(1, _D))


def kernel(pseduo_x, rbf, rsh, edge_index, Wq, Wk, Wp, Wrbf, w_cg, Wv, Wmsg,
           ln_g, ln_b, W1, b1, W2, b2):
    n = pseduo_x.shape[0]
    src = edge_index[0]
    dst = edge_index[1]

    xs = pseduo_x[src]
    xd = pseduo_x[dst]

    wei = _edge_pipeline(xs, xd, rbf, rsh, Wq, Wk, Wp, Wrbf, w_cg, Wv)

    acc = jax.ops.segment_sum(wei, dst, num_segments=n)

    return _node_finalize(acc, Wmsg, ln_g, ln_b, W1, b1, W2, b2)


# speedup vs baseline: 13.9888x; 1.4186x over previous
import functools

import jax
import jax.numpy as jnp
from jax import lax
from jax.experimental import pallas as pl
from jax.experimental.pallas import tpu as pltpu
from jax.experimental.pallas import tpu_sc as plsc

_N = 10000
_E = 320000
_D = 128
_H = 8
_M = 16
_ED = 16
_NB = 16
_EB = 4000
_BN = 1000


_NW = 32
_GC = 80


def _sc_gather_body(x_hbm, src_hbm, dst_hbm, xs_hbm, xd_hbm,
                    sidx, didx, srows, drows, sem):
    wid = lax.axis_index("s") * 2 + lax.axis_index("c")
    per_w = _E // _NW
    base0 = wid * per_w

    def step(i, carry):
        base = base0 + i * _GC
        pltpu.sync_copy(src_hbm.at[pl.ds(base, _GC)], sidx)
        pltpu.sync_copy(dst_hbm.at[pl.ds(base, _GC)], didx)
        g1 = pltpu.async_copy(x_hbm.at[sidx], srows, sem)
        g1.wait()
        g2 = pltpu.async_copy(x_hbm.at[didx], drows, sem)
        g2.wait()
        pltpu.sync_copy(srows, xs_hbm.at[pl.ds(base, _GC)])
        pltpu.sync_copy(drows, xd_hbm.at[pl.ds(base, _GC)])
        return carry

    lax.fori_loop(0, per_w // _GC, step, 0)


def _sc_gather(x, src, dst):
    mesh = plsc.VectorSubcoreMesh(core_axis_name="c", subcore_axis_name="s")
    f = functools.partial(
        pl.kernel,
        out_type=(jax.ShapeDtypeStruct((_E, _D), jnp.float32),
                  jax.ShapeDtypeStruct((_E, _D), jnp.float32)),
        mesh=mesh,
        scratch_types=[
            pltpu.VMEM((_GC,), jnp.int32),
            pltpu.VMEM((_GC,), jnp.int32),
            pltpu.VMEM((_GC, _D), jnp.float32),
            pltpu.VMEM((_GC, _D), jnp.float32),
            pltpu.SemaphoreType.DMA,
        ],
    )(_sc_gather_body)
    return f(x, src, dst)


def _sc_scatter_body(wei_hbm, dstidx_hbm, zeros_hbm, out_hbm,
                     didx, rows, acc):
    c = lax.axis_index("c")
    s = lax.axis_index("s")
    per_core = _E // 2
    per_w = per_core // 16

    @pl.when(s == 0)
    def _():
        pltpu.sync_copy(zeros_hbm, acc)

    plsc.subcore_barrier()
    base0 = c * per_core + s * per_w

    def step(i, carry):
        base = base0 + i * _GC
        pltpu.sync_copy(dstidx_hbm.at[pl.ds(base, _GC)], didx)
        pltpu.sync_copy(wei_hbm.at[pl.ds(base, _GC)], rows)
        pltpu.sync_copy(rows, acc.at[didx], add=True)
        return carry

    lax.fori_loop(0, per_w // _GC, step, 0)
    plsc.subcore_barrier()

    @pl.when(s == 0)
    def _():
        pltpu.sync_copy(acc, out_hbm.at[c])


def _sc_scatter(wei, dst, zeros):
    mesh = plsc.VectorSubcoreMesh(core_axis_name="c", subcore_axis_name="s")
    f = functools.partial(
        pl.kernel,
        out_type=jax.ShapeDtypeStruct((2, _N, _D), jnp.float32),
        mesh=mesh,
        scratch_types=[
            pltpu.VMEM((_GC,), jnp.int32),
            pltpu.VMEM((_GC, _D), jnp.float32),
            pltpu.VMEM_SHARED((_N, _D), jnp.float32),
        ],
    )(_sc_scatter_body)
    return f(wei, dst, zeros)


def _edge_body(xs_ref, xd_ref, rbf_ref, rsh_ref, wq_ref, wk_ref, wp_ref,
               wrbf_ref, wcg_ref, wv_ref, out_ref, ex_ref):
    xs = xs_ref[...]
    xd = xd_ref[...]
    cat = jnp.concatenate([xs, xd], axis=1)
    q = jnp.dot(xs, wq_ref[...])
    k = jnp.dot(cat, wk_ref[...])
    edge = rsh_ref[...] * jnp.dot(rbf_ref[...], wrbf_ref[...])
    sph = edge * jnp.dot(cat, wp_ref[...])
    mixed = wcg_ref[...] * sph * sph
    v = jnp.dot(mixed, wv_ref[...])
    prod = q * k
    heads = [jnp.sum(prod[:, h * _M:(h + 1) * _M], axis=1, keepdims=True)
             for h in range(_H)]
    logits = jnp.concatenate(heads, axis=1) * 0.25
    ex = jnp.exp(jnp.clip(logits, -70.0, 70.0))
    ex_rep = jnp.concatenate(
        [jnp.broadcast_to(ex[:, h:h + 1], (ex.shape[0], _M))
         for h in range(_H)], axis=1)
    out_ref[...] = ex_rep * v
    ex_ref[...] = ex


def _edge_pipeline(xs, xd, rbf, rsh, Wq, Wk, Wp, Wrbf, w_cg, Wv):
    e = xs.shape[0]
    wide = lambda: pl.BlockSpec((_EB, _D), lambda i: (i, 0))
    thin = lambda: pl.BlockSpec((_EB, _ED), lambda i: (i, 0))
    return pl.pallas_call(
        _edge_body,
        grid=(e // _EB,),
        in_specs=[wide(), wide(), thin(), thin(),
                  pl.BlockSpec((_D, _D), lambda i: (0, 0)),
                  pl.BlockSpec((2 * _D, _D), lambda i: (0, 0)),
                  pl.BlockSpec((2 * _D, _ED), lambda i: (0, 0)),
                  pl.BlockSpec((_NB, _ED), lambda i: (0, 0)),
                  pl.BlockSpec((1, _ED), lambda i: (0, 0)),
                  pl.BlockSpec((_ED, _D), lambda i: (0, 0))],
        out_specs=[pl.BlockSpec((_EB, _D), lambda i: (i, 0)),
                   pl.BlockSpec((_EB, _H), lambda i: (i, 0))],
        out_shape=[jax.ShapeDtypeStruct((e, _D), jnp.float32),
                   jax.ShapeDtypeStruct((e, _H), jnp.float32)],
    )(xs, xd, rbf, rsh, Wq, Wk, Wp, Wrbf, w_cg.reshape(1, _ED), Wv)


def _final_body(acc0_ref, acc1_ref, den_ref, wmsg_ref, lng_ref, lnb_ref,
                w1_ref, b1_ref, w2_ref, b2_ref, out_ref):
    msum = acc0_ref[0] + acc1_ref[0]
    den = den_ref[...]
    den_rep = jnp.concatenate(
        [jnp.broadcast_to(den[:, h:h + 1], (den.shape[0], _M))
         for h in range(_H)], axis=1)
    msg = msum / (den_rep + 1e-16)
    msg = jnp.dot(msg, wmsg_ref[...])
    x0 = jnp.abs(msg)
    mu = jnp.mean(x0, axis=-1, keepdims=True)
    var = jnp.mean((x0 - mu) ** 2, axis=-1, keepdims=True)
    x1 = (x0 - mu) / jnp.sqrt(var + 1e-5) * lng_ref[...] + lnb_ref[...]
    x2 = msg / (x0 + 1e-6)
    s = jnp.dot(x1, w1_ref[...]) + b1_ref[...]
    s = s * jax.nn.sigmoid(s)
    s = jnp.dot(s, w2_ref[...]) + b2_ref[...]
    s = s * jax.nn.sigmoid(s)
    out_ref[...] = x2 * s


def _node_finalize(acc_parts, den, Wmsg, ln_g, ln_b, W1, b1, W2, b2):
    n = acc_parts.shape[1]
    rep = pl.BlockSpec((_D, _D), lambda i: (0, 0))
    vec = pl.BlockSpec((1, _D), lambda i: (0, 0))
    return pl.pallas_call(
        _final_body,
        grid=(n // _BN,),
        in_specs=[pl.BlockSpec((1, _BN, _D), lambda i: (0, i, 0)),
                  pl.BlockSpec((1, _BN, _D), lambda i: (1, i, 0)),
                  pl.BlockSpec((_BN, _H), lambda i: (i, 0)),
                  rep, vec, vec, rep, vec, rep, vec],
        out_specs=pl.BlockSpec((_BN, _D), lambda i: (i, 0)),
        out_shape=jax.ShapeDtypeStruct((n, _D), jnp.float32),
    )(acc_parts, acc_parts, den, Wmsg, ln_g.reshape(1, _D),
      ln_b.reshape(1, _D), W1, b1.reshape(1, _D), W2, b2.reshape(1, _D))


def kernel(pseduo_x, rbf, rsh, edge_index, Wq, Wk, Wp, Wrbf, w_cg, Wv, Wmsg,
           ln_g, ln_b, W1, b1, W2, b2):
    src = edge_index[0]
    dst = edge_index[1]

    xs, xd = _sc_gather(pseduo_x, src, dst)

    wei, ex = _edge_pipeline(xs, xd, rbf, rsh, Wq, Wk, Wp, Wrbf, w_cg, Wv)

    zeros = jnp.zeros((_N, _D), jnp.float32)
    acc_parts = _sc_scatter(wei, dst, zeros)
    den = jax.ops.segment_sum(ex, dst, num_segments=_N)

    return _node_finalize(acc_parts, den, Wmsg, ln_g, ln_b, W1, b1, W2, b2)


# speedup vs baseline: 16.0461x; 1.1471x over previous
import functools

import jax
import jax.numpy as jnp
from jax import lax
from jax.experimental import pallas as pl
from jax.experimental.pallas import tpu as pltpu
from jax.experimental.pallas import tpu_sc as plsc

_N = 10000
_E = 320000
_D = 128
_H = 8
_M = 16
_ED = 16
_NB = 16
_EB = 4000
_BN = 1000


_NW = 32
_GC = 80


def _sc_gather_body(x_hbm, src_hbm, dst_hbm, xs_hbm, xd_hbm,
                    sidx, didx, srows, drows, sem):
    wid = lax.axis_index("s") * 2 + lax.axis_index("c")
    per_w = _E // _NW
    base0 = wid * per_w

    def step(i, carry):
        base = base0 + i * _GC
        pltpu.sync_copy(src_hbm.at[pl.ds(base, _GC)], sidx)
        pltpu.sync_copy(dst_hbm.at[pl.ds(base, _GC)], didx)
        g1 = pltpu.async_copy(x_hbm.at[sidx], srows, sem)
        g1.wait()
        g2 = pltpu.async_copy(x_hbm.at[didx], drows, sem)
        g2.wait()
        pltpu.sync_copy(srows, xs_hbm.at[pl.ds(base, _GC)])
        pltpu.sync_copy(drows, xd_hbm.at[pl.ds(base, _GC)])
        return carry

    lax.fori_loop(0, per_w // _GC, step, 0)


def _sc_gather(x, src, dst):
    mesh = plsc.VectorSubcoreMesh(core_axis_name="c", subcore_axis_name="s")
    f = functools.partial(
        pl.kernel,
        out_type=(jax.ShapeDtypeStruct((_E, _D), jnp.float32),
                  jax.ShapeDtypeStruct((_E, _D), jnp.float32)),
        mesh=mesh,
        scratch_types=[
            pltpu.VMEM((_GC,), jnp.int32),
            pltpu.VMEM((_GC,), jnp.int32),
            pltpu.VMEM((_GC, _D), jnp.float32),
            pltpu.VMEM((_GC, _D), jnp.float32),
            pltpu.SemaphoreType.DMA,
        ],
    )(_sc_gather_body)
    return f(x, src, dst)


def _sc_scatter_body(wei_hbm, exw_hbm, dstidx_hbm, zeros_hbm, zeros2_hbm,
                     out_hbm, out2_hbm, didx, rows, exw, stage, ridx,
                     acc, acc2):
    c = lax.axis_index("c")
    s = lax.axis_index("s")
    per_core = _E // 2
    per_w = per_core // 16

    @pl.when(s == 0)
    def _():
        pltpu.sync_copy(zeros_hbm, acc)
        pltpu.sync_copy(zeros2_hbm, acc2)

    z16 = jnp.zeros((16,), jnp.float32)
    for r in range(_GC):
        for cb in range(8):
            stage[r, pl.ds(16 * cb, 16)] = z16

    plsc.subcore_barrier()
    base0 = c * per_core + s * per_w

    def step(i, carry):
        base = base0 + i * _GC
        pltpu.sync_copy(dstidx_hbm.at[pl.ds(base, _GC)], didx)
        pltpu.sync_copy(wei_hbm.at[pl.ds(base, _GC)], rows)
        pltpu.sync_copy(rows, acc.at[didx], add=True)
        pltpu.sync_copy(exw_hbm.at[pl.ds(base, _GC)], exw)
        for g in range(_GC // 16):
            dvec = didx[pl.ds(g * 16, 16)]
            ridx[pl.ds(g * 16, 16)] = lax.shift_right_logical(dvec, 3)
            for j in range(16):
                r = g * 16 + j
                cb = (dvec[j] & 7) * 16
                stage[r, pl.ds(cb, 16)] = exw[r, :]
        pltpu.sync_copy(stage, acc2.at[ridx], add=True)
        for g in range(_GC // 16):
            dvec = didx[pl.ds(g * 16, 16)]
            for j in range(16):
                r = g * 16 + j
                cb = (dvec[j] & 7) * 16
                stage[r, pl.ds(cb, 16)] = z16
        return carry

    lax.fori_loop(0, per_w // _GC, step, 0)
    plsc.subcore_barrier()

    @pl.when(s == 0)
    def _():
        pltpu.sync_copy(acc, out_hbm.at[c])
        pltpu.sync_copy(acc2, out2_hbm.at[c])


def _sc_scatter(wei, exw, dst, zeros, zeros2):
    mesh = plsc.VectorSubcoreMesh(core_axis_name="c", subcore_axis_name="s")
    f = functools.partial(
        pl.kernel,
        out_type=(jax.ShapeDtypeStruct((2, _N, _D), jnp.float32),
                  jax.ShapeDtypeStruct((2, _N // 8, _D), jnp.float32)),
        mesh=mesh,
        scratch_types=[
            pltpu.VMEM((_GC,), jnp.int32),
            pltpu.VMEM((_GC, _D), jnp.float32),
            pltpu.VMEM((_GC, 16), jnp.float32),
            pltpu.VMEM((_GC, _D), jnp.float32),
            pltpu.VMEM((_GC,), jnp.int32),
            pltpu.VMEM_SHARED((_N, _D), jnp.float32),
            pltpu.VMEM_SHARED((_N // 8, _D), jnp.float32),
        ],
    )(_sc_scatter_body)
    return f(wei, exw, dst, zeros, zeros2)


def _edge_body(xs_ref, xd_ref, rbf_ref, rsh_ref, wq_ref, wk_ref, wp_ref,
               wrbf_ref, wcg_ref, wv_ref, out_ref, ex_ref):
    xs = xs_ref[...]
    xd = xd_ref[...]
    cat = jnp.concatenate([xs, xd], axis=1)
    q = jnp.dot(xs, wq_ref[...])
    k = jnp.dot(cat, wk_ref[...])
    edge = rsh_ref[...] * jnp.dot(rbf_ref[...], wrbf_ref[...])
    sph = edge * jnp.dot(cat, wp_ref[...])
    mixed = wcg_ref[...] * sph * sph
    v = jnp.dot(mixed, wv_ref[...])
    prod = q * k
    heads = [jnp.sum(prod[:, h * _M:(h + 1) * _M], axis=1, keepdims=True)
             for h in range(_H)]
    logits = jnp.concatenate(heads, axis=1) * 0.25
    ex = jnp.exp(jnp.clip(logits, -70.0, 70.0))
    ex_rep = jnp.concatenate(
        [jnp.broadcast_to(ex[:, h:h + 1], (ex.shape[0], _M))
         for h in range(_H)], axis=1)
    out_ref[...] = ex_rep * v
    ex_ref[...] = jnp.concatenate([ex, jnp.zeros_like(ex)], axis=1)


def _edge_pipeline(xs, xd, rbf, rsh, Wq, Wk, Wp, Wrbf, w_cg, Wv):
    e = xs.shape[0]
    wide = lambda: pl.BlockSpec((_EB, _D), lambda i: (i, 0))
    thin = lambda: pl.BlockSpec((_EB, _ED), lambda i: (i, 0))
    return pl.pallas_call(
        _edge_body,
        grid=(e // _EB,),
        in_specs=[wide(), wide(), thin(), thin(),
                  pl.BlockSpec((_D, _D), lambda i: (0, 0)),
                  pl.BlockSpec((2 * _D, _D), lambda i: (0, 0)),
                  pl.BlockSpec((2 * _D, _ED), lambda i: (0, 0)),
                  pl.BlockSpec((_NB, _ED), lambda i: (0, 0)),
                  pl.BlockSpec((1, _ED), lambda i: (0, 0)),
                  pl.BlockSpec((_ED, _D), lambda i: (0, 0))],
        out_specs=[pl.BlockSpec((_EB, _D), lambda i: (i, 0)),
                   pl.BlockSpec((_EB, 16), lambda i: (i, 0))],
        out_shape=[jax.ShapeDtypeStruct((e, _D), jnp.float32),
                   jax.ShapeDtypeStruct((e, 16), jnp.float32)],
    )(xs, xd, rbf, rsh, Wq, Wk, Wp, Wrbf, w_cg.reshape(1, _ED), Wv)


def _final_body(acc0_ref, acc1_ref, den0_ref, den1_ref, wmsg_ref, lng_ref,
                lnb_ref, w1_ref, b1_ref, w2_ref, b2_ref, out_ref):
    msum = acc0_ref[0] + acc1_ref[0]
    den = den0_ref[0] + den1_ref[0]
    den_rep = jnp.concatenate(
        [jnp.broadcast_to(den[:, h:h + 1], (den.shape[0], _M))
         for h in range(_H)], axis=1)
    msg = msum / (den_rep + 1e-16)
    msg = jnp.dot(msg, wmsg_ref[...])
    x0 = jnp.abs(msg)
    mu = jnp.mean(x0, axis=-1, keepdims=True)
    var = jnp.mean((x0 - mu) ** 2, axis=-1, keepdims=True)
    x1 = (x0 - mu) / jnp.sqrt(var + 1e-5) * lng_ref[...] + lnb_ref[...]
    x2 = msg / (x0 + 1e-6)
    s = jnp.dot(x1, w1_ref[...]) + b1_ref[...]
    s = s * jax.nn.sigmoid(s)
    s = jnp.dot(s, w2_ref[...]) + b2_ref[...]
    s = s * jax.nn.sigmoid(s)
    out_ref[...] = x2 * s


def _node_finalize(acc_parts, den, Wmsg, ln_g, ln_b, W1, b1, W2, b2):
    n = acc_parts.shape[1]
    rep = pl.BlockSpec((_D, _D), lambda i: (0, 0))
    vec = pl.BlockSpec((1, _D), lambda i: (0, 0))
    return pl.pallas_call(
        _final_body,
        grid=(n // _BN,),
        in_specs=[pl.BlockSpec((1, _BN, _D), lambda i: (0, i, 0)),
                  pl.BlockSpec((1, _BN, _D), lambda i: (1, i, 0)),
                  pl.BlockSpec((1, _BN, _H), lambda i: (0, i, 0)),
                  pl.BlockSpec((1, _BN, _H), lambda i: (1, i, 0)),
                  rep, vec, vec, rep, vec, rep, vec],
        out_specs=pl.BlockSpec((_BN, _D), lambda i: (i, 0)),
        out_shape=jax.ShapeDtypeStruct((n, _D), jnp.float32),
    )(acc_parts, acc_parts, den, den, Wmsg, ln_g.reshape(1, _D),
      ln_b.reshape(1, _D), W1, b1.reshape(1, _D), W2, b2.reshape(1, _D))


def kernel(pseduo_x, rbf, rsh, edge_index, Wq, Wk, Wp, Wrbf, w_cg, Wv, Wmsg,
           ln_g, ln_b, W1, b1, W2, b2):
    src = edge_index[0]
    dst = edge_index[1]

    xs, xd = _sc_gather(pseduo_x, src, dst)

    wei, exw = _edge_pipeline(xs, xd, rbf, rsh, Wq, Wk, Wp, Wrbf, w_cg, Wv)

    zeros = jnp.zeros((_N, _D), jnp.float32)
    zeros2 = jnp.zeros((_N // 8, _D), jnp.float32)
    acc_parts, den_parts = _sc_scatter(wei, exw, dst, zeros, zeros2)
    den = den_parts.reshape(2, _N, 16)[:, :, 0:_H]

    return _node_finalize(acc_parts, den, Wmsg, ln_g, ln_b, W1, b1, W2, b2)


# speedup vs baseline: 16.6521x; 1.0378x over previous
import functools

import jax
import jax.numpy as jnp
from jax import lax
from jax.experimental import pallas as pl
from jax.experimental.pallas import tpu as pltpu
from jax.experimental.pallas import tpu_sc as plsc

_N = 10000
_E = 320000
_D = 128
_H = 8
_M = 16
_ED = 16
_NB = 16
_EB = 4000
_BN = 1000


_NW = 32
_GC = 80


def _sc_gather_body(x_hbm, src_hbm, dst_hbm, xs_hbm, xd_hbm,
                    sidx, didx, srows, drows, sem):
    wid = lax.axis_index("s") * 2 + lax.axis_index("c")
    per_w = _E // _NW
    base0 = wid * per_w

    def step(i, carry):
        base = base0 + i * _GC
        pltpu.sync_copy(src_hbm.at[pl.ds(base, _GC)], sidx)
        pltpu.sync_copy(dst_hbm.at[pl.ds(base, _GC)], didx)
        g1 = pltpu.async_copy(x_hbm.at[sidx], srows, sem)
        g2 = pltpu.async_copy(x_hbm.at[didx], drows, sem)
        g1.wait()
        g2.wait()
        pltpu.sync_copy(srows, xs_hbm.at[pl.ds(base, _GC)])
        pltpu.sync_copy(drows, xd_hbm.at[pl.ds(base, _GC)])
        return carry

    lax.fori_loop(0, per_w // _GC, step, 0)


def _sc_gather(x, src, dst):
    mesh = plsc.VectorSubcoreMesh(core_axis_name="c", subcore_axis_name="s")
    f = functools.partial(
        pl.kernel,
        out_type=(jax.ShapeDtypeStruct((_E, _D), jnp.float32),
                  jax.ShapeDtypeStruct((_E, _D), jnp.float32)),
        mesh=mesh,
        scratch_types=[
            pltpu.VMEM((_GC,), jnp.int32),
            pltpu.VMEM((_GC,), jnp.int32),
            pltpu.VMEM((_GC, _D), jnp.float32),
            pltpu.VMEM((_GC, _D), jnp.float32),
            pltpu.SemaphoreType.DMA,
        ],
    )(_sc_gather_body)
    return f(x, src, dst)


def _sc_scatter_body(wei_hbm, exw_hbm, dstidx_hbm, zeros_hbm, zeros2_hbm,
                     out_hbm, out2_hbm, didx, rows, exw, stage, ridx,
                     acc, acc2):
    c = lax.axis_index("c")
    s = lax.axis_index("s")
    per_core = _E // 2
    per_w = per_core // 16

    @pl.when(s == 0)
    def _():
        pltpu.sync_copy(zeros_hbm, acc)
        pltpu.sync_copy(zeros2_hbm, acc2)

    z16 = jnp.zeros((16,), jnp.float32)
    for r in range(_GC):
        for cb in range(8):
            stage[r, pl.ds(16 * cb, 16)] = z16

    plsc.subcore_barrier()
    base0 = c * per_core + s * per_w

    def step(i, carry):
        base = base0 + i * _GC
        pltpu.sync_copy(dstidx_hbm.at[pl.ds(base, _GC)], didx)
        pltpu.sync_copy(wei_hbm.at[pl.ds(base, _GC)], rows)
        pltpu.sync_copy(rows, acc.at[didx], add=True)
        pltpu.sync_copy(exw_hbm.at[pl.ds(base, _GC)], exw)
        for g in range(_GC // 16):
            dvec = didx[pl.ds(g * 16, 16)]
            ridx[pl.ds(g * 16, 16)] = lax.shift_right_logical(dvec, 3)
            for j in range(16):
                r = g * 16 + j
                cb = (dvec[j] & 7) * 16
                stage[r, pl.ds(cb, 16)] = exw[r, :]
        pltpu.sync_copy(stage, acc2.at[ridx], add=True)
        for g in range(_GC // 16):
            dvec = didx[pl.ds(g * 16, 16)]
            for j in range(16):
                r = g * 16 + j
                cb = (dvec[j] & 7) * 16
                stage[r, pl.ds(cb, 16)] = z16
        return carry

    lax.fori_loop(0, per_w // _GC, step, 0)
    plsc.subcore_barrier()

    @pl.when(s == 0)
    def _():
        pltpu.sync_copy(acc, out_hbm.at[c])
        pltpu.sync_copy(acc2, out2_hbm.at[c])


def _sc_scatter(wei, exw, dst, zeros, zeros2):
    mesh = plsc.VectorSubcoreMesh(core_axis_name="c", subcore_axis_name="s")
    f = functools.partial(
        pl.kernel,
        out_type=(jax.ShapeDtypeStruct((2, _N, _D), jnp.float32),
                  jax.ShapeDtypeStruct((2, _N // 8, _D), jnp.float32)),
        mesh=mesh,
        scratch_types=[
            pltpu.VMEM((_GC,), jnp.int32),
            pltpu.VMEM((_GC, _D), jnp.float32),
            pltpu.VMEM((_GC, 16), jnp.float32),
            pltpu.VMEM((_GC, _D), jnp.float32),
            pltpu.VMEM((_GC,), jnp.int32),
            pltpu.VMEM_SHARED((_N, _D), jnp.float32),
            pltpu.VMEM_SHARED((_N // 8, _D), jnp.float32),
        ],
    )(_sc_scatter_body)
    return f(wei, exw, dst, zeros, zeros2)


def _edge_body(xs_ref, xd_ref, rbf_ref, rsh_ref, wq_ref, wk_ref, wp_ref,
               wrbf_ref, wcg_ref, wv_ref, out_ref, ex_ref):
    xs = xs_ref[...]
    xd = xd_ref[...]
    cat = jnp.concatenate([xs, xd], axis=1)
    q = jnp.dot(xs, wq_ref[...])
    k = jnp.dot(cat, wk_ref[...])
    edge = rsh_ref[...] * jnp.dot(rbf_ref[...], wrbf_ref[...])
    sph = edge * jnp.dot(cat, wp_ref[...])
    mixed = wcg_ref[...] * sph * sph
    v = jnp.dot(mixed, wv_ref[...])
    prod = q * k
    heads = [jnp.sum(prod[:, h * _M:(h + 1) * _M], axis=1, keepdims=True)
             for h in range(_H)]
    logits = jnp.concatenate(heads, axis=1) * 0.25
    ex = jnp.exp(jnp.clip(logits, -70.0, 70.0))
    ex_rep = jnp.concatenate(
        [jnp.broadcast_to(ex[:, h:h + 1], (ex.shape[0], _M))
         for h in range(_H)], axis=1)
    out_ref[...] = ex_rep * v
    ex_ref[...] = jnp.concatenate([ex, jnp.zeros_like(ex)], axis=1)


def _edge_pipeline(xs, xd, rbf, rsh, Wq, Wk, Wp, Wrbf, w_cg, Wv):
    e = xs.shape[0]
    wide = lambda: pl.BlockSpec((_EB, _D), lambda i: (i, 0))
    thin = lambda: pl.BlockSpec((_EB, _ED), lambda i: (i, 0))
    return pl.pallas_call(
        _edge_body,
        grid=(e // _EB,),
        in_specs=[wide(), wide(), thin(), thin(),
                  pl.BlockSpec((_D, _D), lambda i: (0, 0)),
                  pl.BlockSpec((2 * _D, _D), lambda i: (0, 0)),
                  pl.BlockSpec((2 * _D, _ED), lambda i: (0, 0)),
                  pl.BlockSpec((_NB, _ED), lambda i: (0, 0)),
                  pl.BlockSpec((1, _ED), lambda i: (0, 0)),
                  pl.BlockSpec((_ED, _D), lambda i: (0, 0))],
        out_specs=[pl.BlockSpec((_EB, _D), lambda i: (i, 0)),
                   pl.BlockSpec((_EB, 16), lambda i: (i, 0))],
        out_shape=[jax.ShapeDtypeStruct((e, _D), jnp.float32),
                   jax.ShapeDtypeStruct((e, 16), jnp.float32)],
    )(xs, xd, rbf, rsh, Wq, Wk, Wp, Wrbf, w_cg.reshape(1, _ED), Wv)


def _final_body(acc0_ref, acc1_ref, den0_ref, den1_ref, wmsg_ref, lng_ref,
                lnb_ref, w1_ref, b1_ref, w2_ref, b2_ref, out_ref):
    msum = acc0_ref[0] + acc1_ref[0]
    den = den0_ref[0] + den1_ref[0]
    den_rep = jnp.concatenate(
        [jnp.broadcast_to(den[:, h:h + 1], (den.shape[0], _M))
         for h in range(_H)], axis=1)
    msg = msum / (den_rep + 1e-16)
    msg = jnp.dot(msg, wmsg_ref[...])
    x0 = jnp.abs(msg)
    mu = jnp.mean(x0, axis=-1, keepdims=True)
    var = jnp.mean((x0 - mu) ** 2, axis=-1, keepdims=True)
    x1 = (x0 - mu) / jnp.sqrt(var + 1e-5) * lng_ref[...] + lnb_ref[...]
    x2 = msg / (x0 + 1e-6)
    s = jnp.dot(x1, w1_ref[...]) + b1_ref[...]
    s = s * jax.nn.sigmoid(s)
    s = jnp.dot(s, w2_ref[...]) + b2_ref[...]
    s = s * jax.nn.sigmoid(s)
    out_ref[...] = x2 * s


def _node_finalize(acc_parts, den, Wmsg, ln_g, ln_b, W1, b1, W2, b2):
    n = acc_parts.shape[1]
    rep = pl.BlockSpec((_D, _D), lambda i: (0, 0))
    vec = pl.BlockSpec((1, _D), lambda i: (0, 0))
    return pl.pallas_call(
        _final_body,
        grid=(n // _BN,),
        in_specs=[pl.BlockSpec((1, _BN, _D), lambda i: (0, i, 0)),
                  pl.BlockSpec((1, _BN, _D), lambda i: (1, i, 0)),
                  pl.BlockSpec((1, _BN, _H), lambda i: (0, i, 0)),
                  pl.BlockSpec((1, _BN, _H), lambda i: (1, i, 0)),
                  rep, vec, vec, rep, vec, rep, vec],
        out_specs=pl.BlockSpec((_BN, _D), lambda i: (i, 0)),
        out_shape=jax.ShapeDtypeStruct((n, _D), jnp.float32),
    )(acc_parts, acc_parts, den, den, Wmsg, ln_g.reshape(1, _D),
      ln_b.reshape(1, _D), W1, b1.reshape(1, _D), W2, b2.reshape(1, _D))


def kernel(pseduo_x, rbf, rsh, edge_index, Wq, Wk, Wp, Wrbf, w_cg, Wv, Wmsg,
           ln_g, ln_b, W1, b1, W2, b2):
    src = edge_index[0]
    dst = edge_index[1]

    xs, xd = _sc_gather(pseduo_x, src, dst)

    wei, exw = _edge_pipeline(xs, xd, rbf, rsh, Wq, Wk, Wp, Wrbf, w_cg, Wv)

    zeros = jnp.zeros((_N, _D), jnp.float32)
    zeros2 = jnp.zeros((_N // 8, _D), jnp.float32)
    acc_parts, den_parts = _sc_scatter(wei, exw, dst, zeros, zeros2)
    den = den_parts.reshape(2, _N, 16)[:, :, 0:_H]

    return _node_finalize(acc_parts, den, Wmsg, ln_g, ln_b, W1, b1, W2, b2)
